# baseline (speedup 1.0000x reference)
import functools

import jax
import jax.numpy as jnp
from jax import lax
from jax.experimental import pallas as pl
from jax.experimental.pallas import tpu as pltpu
from jax.experimental.pallas import tpu_sc as plsc

NW = 32
CHUNK = 128
RING = 4
ROWS_PER_TILE = 632
N_PAD = 16 * ROWS_PER_TILE


def _tc_transform(x, W, b):
    N, Din = x.shape
    R, _, Dh = W.shape

    def body(x_ref, w_ref, b_ref, out_ref):
        h = jnp.dot(x_ref[...], w_ref[0], preferred_element_type=jnp.float32)
        out_ref[0] = jnp.maximum(h + b_ref[...], 0.0)

    return pl.pallas_call(
        body,
        grid=(R,),
        in_specs=[
            pl.BlockSpec((N, Din), lambda r: (0, 0)),
            pl.BlockSpec((1, Din, Dh), lambda r: (r, 0, 0)),
            pl.BlockSpec((1, Dh), lambda r: (0, 0)),
        ],
        out_specs=pl.BlockSpec((1, N, Dh), lambda r: (r, 0, 0)),
        out_shape=jax.ShapeDtypeStruct((R, N, Dh), jnp.float32),
    )(x, W, b.reshape(1, Dh))


def _tc_transform_sum(parts, W, b, N):
    R, _, Dh = W.shape
    Din = parts.shape[2]

    def body(p_ref, w_ref, b_ref, out_ref):
        h = p_ref[0] + p_ref[1]
        hh = jnp.dot(h, w_ref[0], preferred_element_type=jnp.float32)
        out_ref[0] = jnp.maximum(hh + b_ref[...], 0.0)

    return pl.pallas_call(
        body,
        grid=(R,),
        in_specs=[
            pl.BlockSpec((2, N, Din), lambda r: (0, 0, 0)),
            pl.BlockSpec((1, Din, Dh), lambda r: (r, 0, 0)),
            pl.BlockSpec((1, Dh), lambda r: (0, 0)),
        ],
        out_specs=pl.BlockSpec((1, N, Dh), lambda r: (r, 0, 0)),
        out_shape=jax.ShapeDtypeStruct((R, N, Dh), jnp.float32),
    )(parts, W, b.reshape(1, Dh))


def _tc_sum(parts, N):
    D = parts.shape[2]

    def body(p_ref, out_ref):
        out_ref[...] = p_ref[0] + p_ref[1]

    return pl.pallas_call(
        body,
        grid=(1,),
        in_specs=[pl.BlockSpec((2, N, D), lambda i: (0, 0, 0))],
        out_specs=pl.BlockSpec((N, D), lambda i: (0, 0)),
        out_shape=jax.ShapeDtypeStruct((N, D), jnp.float32),
    )(parts)


def _sc_edge_agg(h_table, idx, zblock, ct, D):
    mesh = plsc.VectorSubcoreMesh(core_axis_name="c", subcore_axis_name="s")

    @functools.partial(
        pl.kernel,
        mesh=mesh,
        out_type=jax.ShapeDtypeStruct((2, N_PAD, D), jnp.float32),
        scratch_types=[
            pltpu.VMEM((RING, 2, CHUNK), jnp.int32),
            pltpu.VMEM((CHUNK, D), jnp.float32),
            pltpu.VMEM((CHUNK, D), jnp.float32),
            pltpu.VMEM_SHARED((N_PAD, D), jnp.float32),
            pltpu.SemaphoreType.DMA,
            pltpu.SemaphoreType.DMA,
            pltpu.SemaphoreType.DMA,
            pltpu.SemaphoreType.DMA,
        ],
    )
    def run(h_hbm, idx_hbm, z_hbm, out_hbm,
            ring, buf0, buf1, acc, semi0, semi1, semg0, semg1):
        cid = lax.axis_index("c")
        sid = lax.axis_index("s")
        base = (sid * 2 + cid) * ct

        pltpu.sync_copy(
            z_hbm, acc.at[pl.ds(sid * ROWS_PER_TILE, ROWS_PER_TILE)])
        plsc.subcore_barrier()

        def fire_idx(j, sem):
            pltpu.async_copy(idx_hbm.at[base + j], ring.at[j % RING], sem)

        def drain_idx(j, sem):
            pltpu.make_async_copy(
                idx_hbm.at[base + j], ring.at[j % RING], sem).wait()

        def fire_g(j, buf, sem):
            pltpu.async_copy(h_hbm.at[ring.at[j % RING, 0]], buf, sem)

        def drain_g(j, buf, sem):
            pltpu.make_async_copy(
                h_hbm.at[ring.at[j % RING, 0]], buf, sem).wait()

        def scatter(j, buf):
            pltpu.sync_copy(buf, acc.at[ring.at[j % RING, 1]], add=True)

        fire_idx(0, semi0)
        fire_idx(1, semi1)
        drain_idx(0, semi0)
        drain_idx(1, semi1)
        fire_idx(2, semi0)
        fire_idx(3, semi1)
        fire_g(0, buf0, semg0)

        def body(p, carry):
            j = 2 * p
            drain_g(j, buf0, semg0)
            fire_g(j + 1, buf1, semg1)
            scatter(j, buf0)
            drain_idx(j + 2, semi0)
            drain_idx(j + 3, semi1)
            fire_idx(j + 4, semi0)
            drain_g(j + 1, buf1, semg1)
            fire_g(j + 2, buf0, semg0)
            scatter(j + 1, buf1)
            fire_idx(j + 5, semi1)
            return carry

        lax.fori_loop(0, ct // 2, body, 0)
        drain_g(ct, buf0, semg0)
        drain_idx(ct + 2, semi0)
        drain_idx(ct + 3, semi1)
        plsc.subcore_barrier()

        pltpu.sync_copy(
            acc.at[pl.ds(sid * ROWS_PER_TILE, ROWS_PER_TILE)],
            out_hbm.at[cid, pl.ds(sid * ROWS_PER_TILE, ROWS_PER_TILE)])

    return run(h_table, idx, zblock)


def kernel(x, edge_index, edge_type, W1, b1, W2, b2):
    N, D = x.shape
    E = edge_index.shape[1]

    src = edge_index[0].astype(jnp.int32)
    dst = edge_index[1].astype(jnp.int32)
    et = edge_type.astype(jnp.int32)

    gidx = et * N + src
    ct = -(-E // (NW * CHUNK))
    ct += ct % 2
    pad = ct * NW * CHUNK - E
    extra = 8 * CHUNK
    gidx = jnp.concatenate([gidx, jnp.zeros((pad + extra,), jnp.int32)])
    didx = jnp.concatenate([dst, jnp.full((pad + extra,), N, jnp.int32)])
    idx = jnp.stack([gidx.reshape(-1, CHUNK), didx.reshape(-1, CHUNK)], axis=1)
    zblock = jnp.zeros((ROWS_PER_TILE, D), jnp.float32)

    H1 = _tc_transform(x, W1, b1).reshape(-1, D)
    parts1 = _sc_edge_agg(H1, idx, zblock, ct, D)
    H2 = _tc_transform_sum(parts1, W2, b2, N).reshape(-1, D)
    parts2 = _sc_edge_agg(H2, idx, zblock, ct, D)
    return _tc_sum(parts2, N)


# speedup vs baseline: 1.1796x; 1.1796x over previous
import functools

import jax
import jax.numpy as jnp
from jax import lax
from jax.experimental import pallas as pl
from jax.experimental.pallas import tpu as pltpu
from jax.experimental.pallas import tpu_sc as plsc

NW = 32
CHUNK = 128
RING = 4
ROWS_PER_TILE = 632
N_PAD = 16 * ROWS_PER_TILE


def _tc_transform(x, W, b):
    N, Din = x.shape
    R, _, Dh = W.shape

    def body(x_ref, w_ref, b_ref, out_ref):
        h = jnp.dot(x_ref[...], w_ref[0], preferred_element_type=jnp.float32)
        out_ref[0] = jnp.maximum(h + b_ref[...], 0.0)

    return pl.pallas_call(
        body,
        grid=(R,),
        in_specs=[
            pl.BlockSpec((N, Din), lambda r: (0, 0)),
            pl.BlockSpec((1, Din, Dh), lambda r: (r, 0, 0)),
            pl.BlockSpec((1, Dh), lambda r: (0, 0)),
        ],
        out_specs=pl.BlockSpec((1, N, Dh), lambda r: (r, 0, 0)),
        out_shape=jax.ShapeDtypeStruct((R, N, Dh), jnp.float32),
    )(x, W, b.reshape(1, Dh))


def _tc_transform_sum(parts, W, b, N):
    R, _, Dh = W.shape
    Din = parts.shape[2]

    def body(p_ref, w_ref, b_ref, out_ref):
        h = p_ref[0] + p_ref[1]
        hh = jnp.dot(h, w_ref[0], preferred_element_type=jnp.float32)
        out_ref[0] = jnp.maximum(hh + b_ref[...], 0.0)

    return pl.pallas_call(
        body,
        grid=(R,),
        in_specs=[
            pl.BlockSpec((2, N, Din), lambda r: (0, 0, 0)),
            pl.BlockSpec((1, Din, Dh), lambda r: (r, 0, 0)),
            pl.BlockSpec((1, Dh), lambda r: (0, 0)),
        ],
        out_specs=pl.BlockSpec((1, N, Dh), lambda r: (r, 0, 0)),
        out_shape=jax.ShapeDtypeStruct((R, N, Dh), jnp.float32),
    )(parts, W, b.reshape(1, Dh))


def _tc_sum(parts, N):
    D = parts.shape[2]

    def body(p_ref, out_ref):
        out_ref[...] = p_ref[0] + p_ref[1]

    return pl.pallas_call(
        body,
        grid=(1,),
        in_specs=[pl.BlockSpec((2, N, D), lambda i: (0, 0, 0))],
        out_specs=pl.BlockSpec((N, D), lambda i: (0, 0)),
        out_shape=jax.ShapeDtypeStruct((N, D), jnp.float32),
    )(parts)


def _sc_edge_agg(h_table, gidx, didx, zblock, ct, D):
    mesh = plsc.VectorSubcoreMesh(core_axis_name="c", subcore_axis_name="s")

    @functools.partial(
        pl.kernel,
        mesh=mesh,
        out_type=jax.ShapeDtypeStruct((2, N_PAD, D), jnp.float32),
        scratch_types=[
            pltpu.VMEM((ct + 8, CHUNK), jnp.int32),
            pltpu.VMEM((RING, CHUNK), jnp.int32),
            pltpu.VMEM((CHUNK, D), jnp.float32),
            pltpu.VMEM((CHUNK, D), jnp.float32),
            pltpu.VMEM_SHARED((N_PAD, D), jnp.float32),
            pltpu.SemaphoreType.DMA,
            pltpu.SemaphoreType.DMA,
            pltpu.SemaphoreType.DMA,
            pltpu.SemaphoreType.DMA,
        ],
    )
    def run(h_hbm, gidx_hbm, didx_hbm, z_hbm, out_hbm,
            gidx_v, dring, buf0, buf1, acc, semd0, semd1, semg0, semg1):
        cid = lax.axis_index("c")
        sid = lax.axis_index("s")
        base = (sid * 2 + cid) * ct

        pltpu.sync_copy(
            z_hbm, acc.at[pl.ds(sid * ROWS_PER_TILE, ROWS_PER_TILE)])
        pltpu.sync_copy(gidx_hbm.at[pl.ds(base, ct + 8)], gidx_v)
        plsc.subcore_barrier()

        def fire_d(j, sem):
            pltpu.async_copy(didx_hbm.at[base + j], dring.at[j % RING], sem)

        def drain_d(j, sem):
            pltpu.make_async_copy(
                didx_hbm.at[base + j], dring.at[j % RING], sem).wait()

        def fire_g(j, buf, sem):
            pltpu.async_copy(h_hbm.at[gidx_v.at[j]], buf, sem)

        def drain_g(j, buf, sem):
            pltpu.make_async_copy(h_hbm.at[gidx_v.at[j]], buf, sem).wait()

        def scatter(j, buf):
            pltpu.sync_copy(buf, acc.at[dring.at[j % RING]], add=True)

        fire_d(0, semd0)
        fire_d(1, semd1)
        drain_d(0, semd0)
        drain_d(1, semd1)
        fire_d(2, semd0)
        fire_d(3, semd1)
        fire_g(0, buf0, semg0)

        def body(p, carry):
            j = 2 * p
            drain_g(j, buf0, semg0)
            fire_g(j + 1, buf1, semg1)
            scatter(j, buf0)
            drain_d(j + 2, semd0)
            fire_d(j + 4, semd0)
            drain_g(j + 1, buf1, semg1)
            fire_g(j + 2, buf0, semg0)
            scatter(j + 1, buf1)
            drain_d(j + 3, semd1)
            fire_d(j + 5, semd1)
            return carry

        lax.fori_loop(0, ct // 2, body, 0)
        drain_g(ct, buf0, semg0)
        drain_d(ct + 2, semd0)
        drain_d(ct + 3, semd1)
        plsc.subcore_barrier()

        pltpu.sync_copy(
            acc.at[pl.ds(sid * ROWS_PER_TILE, ROWS_PER_TILE)],
            out_hbm.at[cid, pl.ds(sid * ROWS_PER_TILE, ROWS_PER_TILE)])

    return run(h_table, gidx, didx, zblock)


def kernel(x, edge_index, edge_type, W1, b1, W2, b2):
    N, D = x.shape
    E = edge_index.shape[1]

    src = edge_index[0].astype(jnp.int32)
    dst = edge_index[1].astype(jnp.int32)
    et = edge_type.astype(jnp.int32)

    gidx = et * N + src
    ct = -(-E // (NW * CHUNK))
    ct += ct % 2
    pad = ct * NW * CHUNK - E
    extra = 8 * CHUNK
    gidx = jnp.concatenate([gidx, jnp.zeros((pad + extra,), jnp.int32)])
    didx = jnp.concatenate([dst, jnp.full((pad + extra,), N, jnp.int32)])
    gidx = gidx.reshape(-1, CHUNK)
    didx = didx.reshape(-1, CHUNK)
    zblock = jnp.zeros((ROWS_PER_TILE, D), jnp.float32)

    H1 = _tc_transform(x, W1, b1).reshape(-1, D)
    parts1 = _sc_edge_agg(H1, gidx, didx, zblock, ct, D)
    H2 = _tc_transform_sum(parts1, W2, b2, N).reshape(-1, D)
    parts2 = _sc_edge_agg(H2, gidx, didx, zblock, ct, D)
    return _tc_sum(parts2, N)
